# Initial kernel scaffold; baseline (speedup 1.0000x reference)
import jax
import jax.numpy as jnp
from jax.experimental import pallas as pl


def kernel(x, edge_index, nodes_of_interest, Wc, bc, Wp, bp, gamma, beta):
    raise NotImplementedError("write your pallas kernel here")


# speedup vs baseline: 22.1693x; 22.1693x over previous
import functools

import jax
import jax.numpy as jnp
from jax import lax
from jax.experimental import pallas as pl
from jax.experimental.pallas import tpu as pltpu
from jax.experimental.pallas import tpu_sc as plsc

_N = 10000
_E = 320000
_D = 128
_DC = 64
_L = 3
_K = 1024
_NC = 2
_NS = 16
_NW = _NC * _NS
_C = 128
_CPW = 80
_G = 16
_EPW = _C * _CPW
_EPAD = _NW * _EPW
_NPAD = 10240
_RPS = _NPAD // _NS
_NB = 10
_BR = _N // _NB

_mesh = plsc.VectorSubcoreMesh(core_axis_name="c", subcore_axis_name="s")


def _zero_vmem_2d(buf, rows):
    def body(i, _):
        for j in range(8):
            buf[i, pl.ds(j * 16, 16)] = jnp.zeros((16,), jnp.float32)
        return _
    lax.fori_loop(0, rows, body, None)


@functools.partial(
    pl.kernel,
    out_type=jax.ShapeDtypeStruct((_NC, _NPAD), jnp.float32),
    mesh=_mesh,
    scratch_types=[
        pltpu.VMEM((_CPW, _C), jnp.int32),
        pltpu.VMEM((_C,), jnp.float32),
        pltpu.VMEM((_RPS,), jnp.float32),
        pltpu.VMEM_SHARED((_NPAD,), jnp.float32),
    ],
)
def _sc_deg(idx_hbm, deg_hbm, idx_v, ones_v, zb_v, acc):
    c = lax.axis_index("c")
    s = lax.axis_index("s")
    w = c * _NS + s
    def zb(i, _):
        zb_v[pl.ds(i * 16, 16)] = jnp.zeros((16,), jnp.float32)
        return _
    lax.fori_loop(0, _RPS // 16, zb, None)
    for j in range(_C // 16):
        ones_v[pl.ds(j * 16, 16)] = jnp.ones((16,), jnp.float32)
    pltpu.sync_copy(zb_v, acc.at[pl.ds(s * _RPS, _RPS)])
    plsc.subcore_barrier()
    pltpu.sync_copy(idx_hbm.at[w], idx_v)
    def step(j, _):
        pltpu.sync_copy(ones_v, acc.at[idx_v.at[j]], add=True)
        return _
    lax.fori_loop(0, _CPW, step, None)
    plsc.subcore_barrier()
    pltpu.sync_copy(acc.at[pl.ds(s * _RPS, _RPS)],
                    deg_hbm.at[c, pl.ds(s * _RPS, _RPS)])


@functools.partial(
    pl.kernel,
    out_type=jax.ShapeDtypeStruct((_NC, _NPAD, _D), jnp.float32),
    mesh=_mesh,
    scratch_types=[
        pltpu.VMEM((_G, _C), jnp.int32),
        pltpu.VMEM((_G, _C), jnp.int32),
        pltpu.VMEM((_C, _D), jnp.float32),
        pltpu.VMEM((_C, _D), jnp.float32),
        pltpu.VMEM_SHARED((_NPAD, _D), jnp.float32),
        pltpu.SemaphoreType.DMA,
        pltpu.SemaphoreType.DMA,
    ],
)
def _sc_scatter(y_hbm, row_hbm, col_hbm, z_hbm,
                row_v, col_v, buf_a, buf_b, acc, sem_a, sem_b):
    c = lax.axis_index("c")
    s = lax.axis_index("s")
    w = c * _NS + s
    _zero_vmem_2d(buf_a, _C)
    for t in range(_RPS // _C):
        pltpu.sync_copy(buf_a, acc.at[pl.ds(s * _RPS + t * _C, _C)])
    plsc.subcore_barrier()
    bufs = (buf_a, buf_b)
    sems = (sem_a, sem_b)
    for g in range(_CPW // _G):
        pltpu.sync_copy(row_hbm.at[w, pl.ds(g * _G, _G)], row_v)
        pltpu.sync_copy(col_hbm.at[w, pl.ds(g * _G, _G)], col_v)
        pltpu.async_copy(y_hbm.at[row_v.at[0]], buf_a, sem_a)
        for t in range(_G):
            cur, nxt = t % 2, (t + 1) % 2
            pltpu.make_async_copy(
                y_hbm.at[row_v.at[t]], bufs[cur], sems[cur]).wait()
            if t + 1 < _G:
                pltpu.async_copy(
                    y_hbm.at[row_v.at[t + 1]], bufs[nxt], sems[nxt])
            pltpu.sync_copy(bufs[cur], acc.at[col_v.at[t]], add=True)
    plsc.subcore_barrier()
    pltpu.sync_copy(acc.at[pl.ds(s * _RPS, _RPS)],
                    z_hbm.at[c, pl.ds(s * _RPS, _RPS)])


@functools.partial(
    pl.kernel,
    out_type=jax.ShapeDtypeStruct((_K, _D), jnp.float32),
    mesh=_mesh,
    scratch_types=[
        pltpu.VMEM((_K // _NW,), jnp.int32),
        pltpu.VMEM((_K // _NW, _D), jnp.float32),
        pltpu.SemaphoreType.DMA,
    ],
)
def _sc_gather(x_hbm, noi_hbm, out_hbm, idx_v, rows_v, sem):
    c = lax.axis_index("c")
    s = lax.axis_index("s")
    w = c * _NS + s
    bpw = _K // _NW
    base = w * bpw
    pltpu.sync_copy(noi_hbm.at[pl.ds(base, bpw)], idx_v)
    pltpu.async_copy(x_hbm.at[idx_v], rows_v, sem).wait()
    pltpu.sync_copy(rows_v, out_hbm.at[pl.ds(base, bpw)])


def _dis_body(dp_ref, out_ref):
    deg = dp_ref[0] + dp_ref[1] + 1.0
    dis = lax.rsqrt(deg)
    r = lax.broadcasted_iota(jnp.int32, (_D, _D), 0)
    cc = lax.broadcasted_iota(jnp.int32, (_D, _D), 1)
    ident = (r == cc).astype(jnp.float32)
    t = lax.dot_general(ident, dis, (((1,), (1,)), ((), ())),
                        preferred_element_type=jnp.float32)
    for j in range(8):
        col = t[:, j:j + 1]
        out_ref[pl.ds(j * _D, _D), :] = jnp.broadcast_to(col, (_D, _D))


def _tc_dis(dp):
    return pl.pallas_call(
        _dis_body,
        grid=(_NPAD // 1024,),
        in_specs=[pl.BlockSpec((_NC, 8, _D), lambda i: (0, i, 0))],
        out_specs=pl.BlockSpec((1024, _D), lambda i: (i, 0)),
        out_shape=jax.ShapeDtypeStruct((_NPAD, _D), jnp.float32),
    )(dp)


def _lin_body(x_ref, dis_ref, w_ref, b_ref, y_ref):
    y = jnp.dot(x_ref[...], w_ref[...], preferred_element_type=jnp.float32)
    y_ref[...] = (y + b_ref[...]) * dis_ref[...]


def _tc_lin(x, dis, w, b):
    return pl.pallas_call(
        _lin_body,
        grid=(_NB,),
        in_specs=[
            pl.BlockSpec((_BR, _D), lambda i: (i, 0)),
            pl.BlockSpec((_BR, _D), lambda i: (i, 0)),
            pl.BlockSpec((_D, _D), lambda i: (0, 0)),
            pl.BlockSpec((1, _D), lambda i: (0, 0)),
        ],
        out_specs=pl.BlockSpec((_BR, _D), lambda i: (i, 0)),
        out_shape=jax.ShapeDtypeStruct((_N, _D), jnp.float32),
    )(x, dis, w, b)


def _layer_body(x_ref, y_ref, z_ref, dis_ref, g_ref, be_ref, w_ref, b_ref,
                xo_ref, yo_ref):
    dis = dis_ref[...]
    h = dis * (z_ref[0] + z_ref[1] + y_ref[...])
    mu = jnp.mean(h, axis=1, keepdims=True)
    ch = h - mu
    var = jnp.mean(ch * ch, axis=1, keepdims=True)
    ln = ch * lax.rsqrt(var + 1e-5) * g_ref[...] + be_ref[...]
    xn = x_ref[...] + jnp.maximum(ln, 0.0)
    xo_ref[...] = xn
    yn = jnp.dot(xn, w_ref[...], preferred_element_type=jnp.float32)
    yo_ref[...] = (yn + b_ref[...]) * dis


def _tc_layer(x, y, z, dis, g, be, w, b):
    return pl.pallas_call(
        _layer_body,
        grid=(_NB,),
        in_specs=[
            pl.BlockSpec((_BR, _D), lambda i: (i, 0)),
            pl.BlockSpec((_BR, _D), lambda i: (i, 0)),
            pl.BlockSpec((_NC, _BR, _D), lambda i: (0, i, 0)),
            pl.BlockSpec((_BR, _D), lambda i: (i, 0)),
            pl.BlockSpec((1, _D), lambda i: (0, 0)),
            pl.BlockSpec((1, _D), lambda i: (0, 0)),
            pl.BlockSpec((_D, _D), lambda i: (0, 0)),
            pl.BlockSpec((1, _D), lambda i: (0, 0)),
        ],
        out_specs=[
            pl.BlockSpec((_BR, _D), lambda i: (i, 0)),
            pl.BlockSpec((_BR, _D), lambda i: (i, 0)),
        ],
        out_shape=[
            jax.ShapeDtypeStruct((_N, _D), jnp.float32),
            jax.ShapeDtypeStruct((_N, _D), jnp.float32),
        ],
    )(x, y, z, dis, g, be, w, b)


def _last_body(x_ref, y_ref, z_ref, dis_ref, g_ref, be_ref, xo_ref):
    dis = dis_ref[...]
    h = dis * (z_ref[0] + z_ref[1] + y_ref[...])
    mu = jnp.mean(h, axis=1, keepdims=True)
    ch = h - mu
    var = jnp.mean(ch * ch, axis=1, keepdims=True)
    ln = ch * lax.rsqrt(var + 1e-5) * g_ref[...] + be_ref[...]
    xo_ref[...] = x_ref[...] + ln


def _tc_last(x, y, z, dis, g, be):
    return pl.pallas_call(
        _last_body,
        grid=(_NB,),
        in_specs=[
            pl.BlockSpec((_BR, _D), lambda i: (i, 0)),
            pl.BlockSpec((_BR, _D), lambda i: (i, 0)),
            pl.BlockSpec((_NC, _BR, _D), lambda i: (0, i, 0)),
            pl.BlockSpec((_BR, _D), lambda i: (i, 0)),
            pl.BlockSpec((1, _D), lambda i: (0, 0)),
            pl.BlockSpec((1, _D), lambda i: (0, 0)),
        ],
        out_specs=pl.BlockSpec((_BR, _D), lambda i: (i, 0)),
        out_shape=jax.ShapeDtypeStruct((_N, _D), jnp.float32),
    )(x, y, z, dis, g, be)


def kernel(x, edge_index, nodes_of_interest, Wc, bc, Wp, bp, gamma, beta):
    f32 = jnp.float32
    Wf = jnp.zeros((_L, _D, _D), f32)
    Wf = Wf.at[:, :_DC, :_DC].set(jnp.swapaxes(Wc, 1, 2))
    Wf = Wf.at[:, _DC:, _DC:].set(jnp.swapaxes(Wp, 1, 2))
    bf = jnp.concatenate([bc, bp], axis=1).reshape(_L, 1, _D)
    g2 = gamma.reshape(_L, 1, _D)
    b2 = beta.reshape(_L, 1, _D)

    row = edge_index[0]
    col = edge_index[1]
    npad = _EPAD - _E
    padi = jnp.arange(npad, dtype=jnp.int32)
    pad_dst = _N + padi % (_NPAD - _N)
    row_p = jnp.concatenate([row, padi % _N]).reshape(_NW, _CPW, _C)
    col_p = jnp.concatenate([col, pad_dst]).reshape(_NW, _CPW, _C)
    deg_dst = jnp.concatenate([row, pad_dst]).reshape(_NW, _CPW, _C)

    dp = _sc_deg(deg_dst)
    dis = _tc_dis(dp.reshape(_NC, _NPAD // _D, _D))

    y = _tc_lin(x, dis, Wf[0], bf[0])
    z = _sc_scatter(y, row_p, col_p)
    x1, y1 = _tc_layer(x, y, z, dis, g2[0], b2[0], Wf[1], bf[1])
    z1 = _sc_scatter(y1, row_p, col_p)
    x2, y2 = _tc_layer(x1, y1, z1, dis, g2[1], b2[1], Wf[2], bf[2])
    z2 = _sc_scatter(y2, row_p, col_p)
    x3 = _tc_last(x2, y2, z2, dis, g2[2], b2[2])
    return _sc_gather(x3, nodes_of_interest)
